# Initial kernel scaffold; baseline (speedup 1.0000x reference)
import jax
import jax.numpy as jnp
from jax.experimental import pallas as pl


def kernel(X, X_w, W):
    raise NotImplementedError("write your pallas kernel here")


# speedup vs baseline: 2.1768x; 2.1768x over previous
import functools

import jax
import jax.numpy as jnp
from jax import lax
from jax.experimental import pallas as pl
from jax.experimental.pallas import tpu as pltpu
from jax.experimental.pallas import tpu_sc as plsc

_INFO = plsc.get_sparse_core_info()
_NC = _INFO.num_cores
_NS = _INFO.num_subcores
_NW = _NC * _NS
_LANES = _INFO.num_lanes


@functools.lru_cache(maxsize=None)
def _make_embedding_bag(B, H, D, V):
    G = 2
    assert B % (_NW * G) == 0
    S = B // (_NW * G)
    R = G * H
    assert R <= 128
    BPW = B // _NW
    KD = D // _LANES

    mesh = plsc.VectorSubcoreMesh(core_axis_name="c", subcore_axis_name="s")

    @functools.partial(
        pl.kernel,
        mesh=mesh,
        compiler_params=pltpu.CompilerParams(use_tc_tiling_on_sc=False),
        out_type=jax.ShapeDtypeStruct((B, D), jnp.float32),
        scratch_types=[
            pltpu.VMEM((S, R), jnp.int32),
            pltpu.VMEM((S, R), jnp.float32),
            pltpu.VMEM((R, D), jnp.float32),
            pltpu.VMEM((BPW, D), jnp.float32),
            pltpu.SemaphoreType.DMA,
        ],
    )
    def bag(table_hbm, idx_hbm, wgt_hbm, out_hbm,
            idx_v, wgt_v, rows_v, out_v, sem):
        wid = lax.axis_index("s") * _NC + lax.axis_index("c")
        pltpu.sync_copy(idx_hbm.at[wid], idx_v)
        pltpu.sync_copy(wgt_hbm.at[wid], wgt_v)

        def step(s, carry):
            pltpu.async_copy(table_hbm.at[idx_v.at[s]], rows_v, sem).wait()
            for j in range(G):
                base = j * H
                chunk_offs = [0, 16, 32, H - _LANES]
                wvecs = [wgt_v[s, pl.ds(base + o, _LANES)] for o in chunk_offs]

                def wlane(n):
                    if n < 48:
                        return wvecs[n // 16][n % 16]
                    return wvecs[3][n - (H - _LANES)]

                acc = [jnp.zeros((_LANES,), jnp.float32) for _ in range(KD)]
                for n in range(H):
                    p = j * H + n
                    w = wlane(n)
                    for k in range(KD):
                        acc[k] = acc[k] + rows_v[p, pl.ds(k * _LANES, _LANES)] * w
                b_local = s * G + j
                for k in range(KD):
                    out_v[b_local, pl.ds(k * _LANES, _LANES)] = acc[k]
            return carry

        lax.fori_loop(0, S, step, 0)
        pltpu.sync_copy(out_v, out_hbm.at[pl.ds(wid * BPW, BPW)])

    return bag


def kernel(X, X_w, W):
    B, H = X.shape
    V, D = W.shape
    G = 2
    S = B // (_NW * G)
    Xr = X.astype(jnp.int32).reshape(_NW, S, G * H)
    Wr = X_w.astype(jnp.float32).reshape(_NW, S, G * H)
    return _make_embedding_bag(B, H, D, V)(W, Xr, Wr)


# speedup vs baseline: 2.4988x; 1.1480x over previous
import functools

import jax
import jax.numpy as jnp
from jax import lax
from jax.experimental import pallas as pl
from jax.experimental.pallas import tpu as pltpu
from jax.experimental.pallas import tpu_sc as plsc

_INFO = plsc.get_sparse_core_info()
_NC = _INFO.num_cores
_NS = _INFO.num_subcores
_NW = _NC * _NS
_LANES = _INFO.num_lanes
_NBUF = 4


@functools.lru_cache(maxsize=None)
def _make_embedding_bag(B, H, D, V):
    G = 2
    assert B % (_NW * G) == 0
    S = B // (_NW * G)
    R = G * H
    assert R <= 128
    BPW = B // _NW
    KD = D // _LANES

    mesh = plsc.VectorSubcoreMesh(core_axis_name="c", subcore_axis_name="s")

    @functools.partial(
        pl.kernel,
        mesh=mesh,
        compiler_params=pltpu.CompilerParams(use_tc_tiling_on_sc=False),
        out_type=jax.ShapeDtypeStruct((B, D), jnp.float32),
        scratch_types=[
            pltpu.VMEM((S, R), jnp.int32),
            pltpu.VMEM((S, R), jnp.float32),
            [pltpu.VMEM((R, D), jnp.float32)] * _NBUF,
            pltpu.VMEM((BPW, D), jnp.float32),
            [pltpu.SemaphoreType.DMA] * _NBUF,
        ],
    )
    def bag(table_hbm, idx_hbm, wgt_hbm, out_hbm,
            idx_v, wgt_v, rows_bufs, out_v, sems):
        wid = lax.axis_index("s") * _NC + lax.axis_index("c")
        pltpu.sync_copy(idx_hbm.at[wid], idx_v)
        pltpu.sync_copy(wgt_hbm.at[wid], wgt_v)

        def gstart(s, b):
            pltpu.async_copy(table_hbm.at[idx_v.at[s]], rows_bufs[b], sems[b])

        def gwait(s, b):
            pltpu.make_async_copy(
                table_hbm.at[idx_v.at[s]], rows_bufs[b], sems[b]).wait()

        def compute(s, rows_v):
            for j in range(G):
                base = j * H
                chunk_offs = [0, 16, 32, H - _LANES]
                wvecs = [wgt_v[s, pl.ds(base + o, _LANES)] for o in chunk_offs]

                def wlane(n):
                    if n < 48:
                        return wvecs[n // 16][n % 16]
                    return wvecs[3][n - (H - _LANES)]

                acc = [rows_v[j * H, pl.ds(k * _LANES, _LANES)] * wlane(0)
                       for k in range(KD)]
                for n in range(1, H):
                    p = j * H + n
                    w = wlane(n)
                    for k in range(KD):
                        acc[k] = acc[k] + rows_v[p, pl.ds(k * _LANES, _LANES)] * w
                b_local = s * G + j
                for k in range(KD):
                    out_v[b_local, pl.ds(k * _LANES, _LANES)] = acc[k]

        for i in range(_NBUF - 1):
            gstart(i, i)

        def round_(t, carry):
            s0 = t * _NBUF
            for b in range(_NBUF):
                s = s0 + b
                gwait(s, b)
                compute(s, rows_bufs[b])
                nxt = s + _NBUF - 1

                @pl.when(nxt < S)
                def _():
                    gstart(nxt, (b + _NBUF - 1) % _NBUF)
            return carry

        lax.fori_loop(0, S // _NBUF, round_, 0)
        pltpu.sync_copy(out_v, out_hbm.at[pl.ds(wid * BPW, BPW)])

    return bag


def kernel(X, X_w, W):
    B, H = X.shape
    V, D = W.shape
    G = 2
    S = B // (_NW * G)
    Xr = X.astype(jnp.int32).reshape(_NW, S, G * H)
    Wr = X_w.astype(jnp.float32).reshape(_NW, S, G * H)
    return _make_embedding_bag(B, H, D, V)(W, Xr, Wr)
